# initial kernel scaffold (unmeasured)
import jax
import jax.numpy as jnp
from jax import lax
from jax.experimental import pallas as pl
from jax.experimental.pallas import tpu as pltpu

N_DEV = 8
N_LAYERS = 3


def kernel(x, Win0, Wout0, Win1, Wout1, Win2, Wout2):
    b, d = x.shape
    rows_out = b // N_DEV

    def body(x_ref, win0_ref, wout0_ref, win1_ref, wout1_ref, win2_ref,
             wout2_ref, out_ref, comm_ref, acc_ref, send_sems, recv_sems):
        my = lax.axis_index("i")
        wins = [win0_ref, win1_ref, win2_ref]
        wouts = [wout0_ref, wout1_ref, wout2_ref]

        xv = x_ref[...].astype(jnp.bfloat16)
        for r in range(N_LAYERS):
            h = jnp.dot(xv, wins[r][...].astype(jnp.bfloat16),
                        preferred_element_type=jnp.float32)
            h = jnp.maximum(h, 0.0).astype(jnp.bfloat16)
            partial = jnp.dot(h, wouts[r][...].astype(jnp.bfloat16),
                              preferred_element_type=jnp.float32)

            comm_ref[r, pl.ds(my, 1)] = partial.astype(jnp.bfloat16)[None]
            for k in range(1, N_DEV):
                tgt = lax.rem(my + k, N_DEV)
                rdma = pltpu.make_async_remote_copy(
                    src_ref=comm_ref.at[r, pl.ds(my, 1)],
                    dst_ref=comm_ref.at[r, pl.ds(my, 1)],
                    send_sem=send_sems.at[r, k - 1],
                    recv_sem=recv_sems.at[r, pl.ds(my, 1)],
                    device_id=(tgt,),
                    device_id_type=pl.DeviceIdType.MESH,
                )
                rdma.start()

            for k in range(1, N_DEV):
                src = lax.rem(my + k, N_DEV)
                recv = pltpu.make_async_remote_copy(
                    src_ref=comm_ref.at[r, pl.ds(src, 1)],
                    dst_ref=comm_ref.at[r, pl.ds(src, 1)],
                    send_sem=send_sems.at[r, k - 1],
                    recv_sem=recv_sems.at[r, pl.ds(src, 1)],
                    device_id=(src,),
                    device_id_type=pl.DeviceIdType.MESH,
                )
                recv.wait_recv()

            total = jnp.sum(comm_ref[r].astype(jnp.float32), axis=0)

            for k in range(1, N_DEV):
                tgt = lax.rem(my + k, N_DEV)
                send = pltpu.make_async_remote_copy(
                    src_ref=comm_ref.at[r, pl.ds(my, 1)],
                    dst_ref=comm_ref.at[r, pl.ds(my, 1)],
                    send_sem=send_sems.at[r, k - 1],
                    recv_sem=recv_sems.at[r, pl.ds(my, 1)],
                    device_id=(tgt,),
                    device_id_type=pl.DeviceIdType.MESH,
                )
                send.wait_send()

            if r < N_LAYERS - 1:
                xv = total.astype(jnp.bfloat16)
            else:
                acc_ref[...] = total
                out_ref[...] = acc_ref[pl.ds(my * rows_out, rows_out), :]

    return pl.pallas_call(
        body,
        out_shape=jax.ShapeDtypeStruct((rows_out, d), jnp.float32),
        in_specs=[pl.BlockSpec(memory_space=pltpu.VMEM)] * 7,
        out_specs=pl.BlockSpec(memory_space=pltpu.VMEM),
        scratch_shapes=[
            pltpu.VMEM((N_LAYERS, N_DEV, b, d), jnp.bfloat16),
            pltpu.VMEM((b, d), jnp.float32),
            pltpu.SemaphoreType.DMA((N_LAYERS, N_DEV - 1)),
            pltpu.SemaphoreType.DMA((N_LAYERS, N_DEV)),
        ],
        compiler_params=pltpu.CompilerParams(collective_id=0),
    )(x, Win0, Wout0, Win1, Wout1, Win2, Wout2)


# baseline (device time: 13201 ns/iter reference)
import jax
import jax.numpy as jnp
from jax import lax
from jax.experimental import pallas as pl
from jax.experimental.pallas import tpu as pltpu

N_DEV = 8
N_LAYERS = 3


def kernel(x, Win0, Wout0, Win1, Wout1, Win2, Wout2):
    b, d = x.shape
    rows_out = b // N_DEV

    def body(x_ref, win0_ref, wout0_ref, win1_ref, wout1_ref, win2_ref,
             wout2_ref, out_ref, comm_ref, acc_ref, send_sems, recv_sems):
        my = lax.axis_index("i")
        wins = [win0_ref, win1_ref, win2_ref]
        wouts = [wout0_ref, wout1_ref, wout2_ref]

        xv = x_ref[...].astype(jnp.bfloat16)
        for r in range(N_LAYERS):
            h = jnp.dot(xv, wins[r][...].astype(jnp.bfloat16),
                        preferred_element_type=jnp.float32)
            h = jnp.maximum(h, 0.0).astype(jnp.bfloat16)
            partial = jnp.dot(h, wouts[r][...].astype(jnp.bfloat16),
                              preferred_element_type=jnp.float32)

            comm_ref[r, pl.ds(my, 1)] = partial.astype(jnp.bfloat16)[None]
            total = jnp.sum(comm_ref[r].astype(jnp.float32), axis=0)

            if r < N_LAYERS - 1:
                xv = total.astype(jnp.bfloat16)
            else:
                acc_ref[...] = total
                out_ref[...] = acc_ref[pl.ds(my * rows_out, rows_out), :]

    return pl.pallas_call(
        body,
        out_shape=jax.ShapeDtypeStruct((rows_out, d), jnp.float32),
        in_specs=[pl.BlockSpec(memory_space=pltpu.VMEM)] * 7,
        out_specs=pl.BlockSpec(memory_space=pltpu.VMEM),
        scratch_shapes=[
            pltpu.VMEM((N_LAYERS, N_DEV, b, d), jnp.bfloat16),
            pltpu.VMEM((b, d), jnp.float32),
            pltpu.SemaphoreType.DMA((N_LAYERS, N_DEV - 1)),
            pltpu.SemaphoreType.DMA((N_LAYERS, N_DEV)),
        ],
    )(x, Win0, Wout0, Win1, Wout1, Win2, Wout2)
